# baseline (speedup 1.0000x reference)
import functools

import jax
import jax.numpy as jnp
from jax import lax
from jax.experimental import pallas as pl
from jax.experimental.pallas import tpu as pltpu
from jax.experimental.pallas import tpu_sc as plsc

N = 10000
E = 160000
NS = 16
EPT = E // NS
K = 50
NCH = EPT // K
EG = EPT // 16
RPT = N // NS
RQ = 624

_SC_PARAMS = pltpu.CompilerParams(use_tc_tiling_on_sc=False,
                                  needs_layout_passes=False)
_MESH_KW = dict(core_axis_name="c", subcore_axis_name="s",
                num_cores=2, num_subcores=NS)


@functools.cache
def _alpha_pass():
    ewa = 5008
    ewl = E - 31 * ewa

    @functools.partial(
        pl.kernel,
        mesh=plsc.VectorSubcoreMesh(**_MESH_KW),
        out_type=jax.ShapeDtypeStruct((E,), jnp.float32),
        compiler_params=_SC_PARAMS,
        scratch_types=[
            pltpu.VMEM((N,), jnp.float32),
            pltpu.VMEM((N,), jnp.float32),
            pltpu.VMEM((ewa,), jnp.int32),
            pltpu.VMEM((ewa,), jnp.int32),
            pltpu.VMEM((ewa,), jnp.float32),
        ],
    )
    def alpha(asn_hbm, adn_hbm, src_hbm, dst_hbm, w_hbm,
              asn_v, adn_v, src_f, dst_f, w_f):
        cid = lax.axis_index("c")
        sid = lax.axis_index("s")
        wid = sid * 2 + cid
        base = wid * ewa

        pltpu.sync_copy(asn_hbm, asn_v)
        pltpu.sync_copy(adn_hbm, adn_v)

        @pl.when(wid < 31)
        def _():
            pltpu.sync_copy(src_hbm.at[pl.ds(base, ewa)], src_f)
            pltpu.sync_copy(dst_hbm.at[pl.ds(base, ewa)], dst_f)

        @pl.when(wid == 31)
        def _():
            pltpu.sync_copy(src_hbm.at[pl.ds(base, ewl)],
                            src_f.at[pl.ds(0, ewl)])
            pltpu.sync_copy(dst_hbm.at[pl.ds(base, ewl)],
                            dst_f.at[pl.ds(0, ewl)])

        def wgrp(g, _):
            sv = src_f[pl.ds(g * 16, 16)]
            dv = dst_f[pl.ds(g * 16, 16)]
            al = plsc.load_gather(asn_v, [sv]) + plsc.load_gather(
                adn_v, [dv])
            w_f[pl.ds(g * 16, 16)] = jnp.exp(1.0 / (1.0 + jnp.exp(-al)))
            return 0

        ng = jnp.where(wid == 31, ewl // 16, ewa // 16)
        lax.fori_loop(0, ng, wgrp, 0)

        @pl.when(wid < 31)
        def _():
            pltpu.sync_copy(w_f, w_hbm.at[pl.ds(base, ewa)])

        @pl.when(wid == 31)
        def _():
            pltpu.sync_copy(w_f.at[pl.ds(0, ewl)],
                            w_hbm.at[pl.ds(base, ewl)])

    return alpha


@functools.cache
def _make_agg_pass(dh):
    @functools.partial(
        pl.kernel,
        mesh=plsc.VectorSubcoreMesh(**_MESH_KW),
        out_type=[jax.ShapeDtypeStruct((2 * N, dh), jnp.float32),
                  jax.ShapeDtypeStruct((N,), jnp.float32)],
        compiler_params=_SC_PARAMS,
        scratch_types=[
            pltpu.VMEM((NCH, K), jnp.int32),
            pltpu.VMEM((NCH, K), jnp.int32),
            pltpu.VMEM((K, dh), jnp.float32),
            pltpu.VMEM((K, dh), jnp.float32),
            pltpu.VMEM((K, dh), jnp.float32),
            pltpu.VMEM((K,), jnp.float32),
            pltpu.VMEM((K,), jnp.float32),
            pltpu.VMEM((K,), jnp.float32),
            pltpu.VMEM((640,), jnp.float32),
            pltpu.VMEM_SHARED((N, dh), jnp.float32),
            pltpu.VMEM_SHARED((N,), jnp.float32),
            pltpu.SemaphoreType.DMA,
            pltpu.SemaphoreType.DMA,
            pltpu.SemaphoreType.DMA,
            pltpu.SemaphoreType.DMA,
            pltpu.SemaphoreType.DMA,
            pltpu.SemaphoreType.DMA,
        ],
    )
    def agg(h_hbm, w_hbm, src_hbm, dst_hbm, out_hbm, den_hbm,
            src_v, dst_v, rows_a, rows_b, rows_c, w_a, w_b, w_c, zbuf,
            acc, den_acc, g_a, g_b, g_c, s_a, s_b, s_c):
        cid = lax.axis_index("c")
        sid = lax.axis_index("s")

        pltpu.sync_copy(src_hbm.at[sid], src_v)
        pltpu.sync_copy(dst_hbm.at[sid], dst_v)

        coff = cid * N

        def offs(j, _):
            for g in range(K // 16):
                src_v[j, pl.ds(g * 16, 16)] = (
                    src_v[j, pl.ds(g * 16, 16)] + coff)
            return 0

        lax.fori_loop(0, NCH, offs, 0)

        zeros16 = jnp.zeros((16,), jnp.float32)

        def zrow(r, _):
            for t in range(dh // 16):
                rows_a[r, pl.ds(t * 16, 16)] = zeros16
            return 0

        lax.fori_loop(0, K, zrow, 0)

        def zb(r, _):
            zbuf[pl.ds(r * 16, 16)] = zeros16
            return 0

        lax.fori_loop(0, 640 // 16, zb, 0)

        nz = RPT // K
        for b in range(nz):
            pltpu.sync_copy(rows_a, acc.at[pl.ds(sid * RPT + b * K, K), :])
        rem = RPT - nz * K
        pltpu.sync_copy(rows_a.at[pl.ds(0, rem), :],
                        acc.at[pl.ds(sid * RPT + nz * K, rem), :])
        pltpu.sync_copy(zbuf.at[pl.ds(0, RQ)],
                        den_acc.at[pl.ds(sid * RQ, RQ)])

        @pl.when(sid == NS - 1)
        def _():
            pltpu.sync_copy(zbuf.at[pl.ds(0, N - NS * RQ)],
                            den_acc.at[pl.ds(NS * RQ, N - NS * RQ)])

        plsc.subcore_barrier()

        wrow = sid * NCH

        def start_gather(j, rows_x, w_x, g_x):
            pltpu.async_copy(h_hbm.at[src_v.at[j]], rows_x, g_x)
            pltpu.async_copy(w_hbm.at[wrow + j], w_x, g_x)

        def wait_gather(j, rows_x, w_x, g_x):
            pltpu.make_async_copy(h_hbm.at[src_v.at[j]], rows_x, g_x).wait()
            pltpu.make_async_copy(w_hbm.at[wrow + j], w_x, g_x).wait()

        def scale(rows_x, w_x):
            def srow(e, _):
                wb = plsc.load_gather(
                    w_x, [jnp.full((16,), e, jnp.int32)])
                for t in range(dh // 16):
                    rows_x[e, pl.ds(t * 16, 16)] = (
                        rows_x[e, pl.ds(t * 16, 16)] * wb)
                return 0

            lax.fori_loop(0, K, srow, 0, unroll=4)

        def start_scatter(j, rows_x, w_x, s_x):
            pltpu.async_copy(rows_x, acc.at[dst_v.at[j]], s_x, add=True)
            pltpu.async_copy(w_x, den_acc.at[dst_v.at[j]], s_x, add=True)

        def wait_scatter(j, rows_x, w_x, s_x):
            pltpu.make_async_copy(
                rows_x, acc.at[dst_v.at[j]], s_x).wait()
            pltpu.make_async_copy(
                w_x, den_acc.at[dst_v.at[j]], s_x).wait()

        bufs = [(rows_a, w_a, g_a, s_a),
                (rows_b, w_b, g_b, s_b),
                (rows_c, w_c, g_c, s_c)]

        def slot(j, t):
            rows_x, w_x, g_x, s_x = bufs[t]
            wait_gather(j, rows_x, w_x, g_x)
            scale(rows_x, w_x)
            start_scatter(j, rows_x, w_x, s_x)
            if isinstance(j, int):
                if j + 2 < NCH:
                    ry, wy, gy, sy = bufs[(t + 2) % 3]
                    if j >= 1:
                        wait_scatter(j - 1, ry, wy, sy)
                    start_gather(j + 2, ry, wy, gy)
            else:
                @pl.when(j + 2 < NCH)
                def _():
                    ry, wy, gy, sy = bufs[(t + 2) % 3]

                    @pl.when(j >= 1)
                    def _():
                        wait_scatter(j - 1, ry, wy, sy)

                    start_gather(j + 2, ry, wy, gy)

        start_gather(0, rows_a, w_a, g_a)
        start_gather(1, rows_b, w_b, g_b)

        nfull = NCH // 3

        def body(m, _):
            j0 = 3 * m
            slot(j0, 0)
            slot(j0 + 1, 1)
            slot(j0 + 2, 2)
            return 0

        lax.fori_loop(0, nfull, body, 0)
        for j in range(3 * nfull, NCH):
            slot(j, j % 3)
        for j in range(NCH - 3, NCH):
            rx, wx, gx, sx = bufs[j % 3]
            wait_scatter(j, rx, wx, sx)
        plsc.subcore_barrier()
        pltpu.sync_copy(acc.at[pl.ds(sid * RPT, RPT), :],
                        out_hbm.at[pl.ds(coff + sid * RPT, RPT), :])

        @pl.when(cid == 0)
        def _():
            pltpu.sync_copy(den_acc.at[pl.ds(sid * RQ, RQ)],
                            den_hbm.at[pl.ds(sid * RQ, RQ)])

            @pl.when(sid == NS - 1)
            def _():
                pltpu.sync_copy(den_acc.at[pl.ds(NS * RQ, N - NS * RQ)],
                                den_hbm.at[pl.ds(NS * RQ, N - NS * RQ)])

    return agg


def _hext(h, d):
    dh = d // 2
    return jnp.concatenate([h[:, :dh], h[:, dh:]], axis=0)


def _unpack_norm(acc, den, d):
    dh = d // 2
    inv = 1.0 / (den + 1e-16)
    return jnp.concatenate([acc[:N, :dh] * inv, acc[N:, :dh] * inv], axis=1)


def _bn_body(y, g, b):
    mu = jnp.mean(y, axis=0, keepdims=True)
    yc = y - mu
    var = jnp.mean(yc * yc, axis=0, keepdims=True)
    return yc * lax.rsqrt(var + 1e-5) * g + b


def _leaky(x, slope):
    return jnp.where(x >= 0, x, slope * x)


def _tc_first(x_ref, w_ref, as_ref, ad_ref, hext_ref, asn_ref, adn_ref):
    h = jnp.dot(x_ref[...], w_ref[...], preferred_element_type=jnp.float32)
    asn_ref[...] = jnp.sum(h * as_ref[...], axis=1, keepdims=True)
    adn_ref[...] = jnp.sum(h * ad_ref[...], axis=1, keepdims=True)
    hext_ref[...] = _hext(h, w_ref.shape[1])


def _make_tc_mid(d, slope):
    def body(acc_ref, den_ref, g_ref, b_ref, w_ref, as_ref, ad_ref,
             hext_ref, asn_ref, adn_ref):
        y = _unpack_norm(acc_ref[...], den_ref[...], d)
        ybn = _bn_body(y, g_ref[...], b_ref[...])
        if slope is not None:
            ybn = _leaky(ybn, slope)
        h = jnp.dot(ybn, w_ref[...], preferred_element_type=jnp.float32)
        asn_ref[...] = jnp.sum(h * as_ref[...], axis=1, keepdims=True)
        adn_ref[...] = jnp.sum(h * ad_ref[...], axis=1, keepdims=True)
        hext_ref[...] = _hext(h, w_ref.shape[1])
    return body


def _tc_mid2_heads(acc_ref, den_ref, g_ref, b_ref, w_ref, as_ref, ad_ref,
                   tw1_ref, tb1_ref, tw2_ref, tb2_ref,
                   cw1_ref, cb1_ref, cw2_ref, cb2_ref,
                   hext_ref, asn_ref, adn_ref, tp_ref, cl_ref):
    z = _bn_body(_unpack_norm(acc_ref[...], den_ref[...], 128),
                 g_ref[...], b_ref[...])
    h = jnp.dot(z, w_ref[...], preferred_element_type=jnp.float32)
    asn_ref[...] = jnp.sum(h * as_ref[...], axis=1, keepdims=True)
    adn_ref[...] = jnp.sum(h * ad_ref[...], axis=1, keepdims=True)
    hext_ref[...] = _hext(h, w_ref.shape[1])
    t = _leaky(jnp.dot(z, tw1_ref[...]) + tb1_ref[...], 0.01)
    tp_ref[...] = jax.nn.sigmoid(jnp.dot(t, tw2_ref[...]) + tb2_ref[...])
    c = _leaky(jnp.dot(z, cw1_ref[...]) + cb1_ref[...], 0.01)
    cl_ref[...] = jnp.dot(c, cw2_ref[...]) + cb2_ref[...]


def _tc_final(acc_ref, den_ref, g_ref, b_ref, out_ref):
    out_ref[...] = _bn_body(_unpack_norm(acc_ref[...], den_ref[...], 256),
                            g_ref[...], b_ref[...])


def _sds(shape):
    return jax.ShapeDtypeStruct(shape, jnp.float32)


def _edge_layer(hext, asn, adn, src2, dst2, src3, dst3, dh):
    w = _alpha_pass()(asn.reshape(N), adn.reshape(N), src2, dst2)
    acc, den = _make_agg_pass(dh)(hext, w.reshape(NS * NCH, K), src3, dst3)
    return acc, den.reshape(N, 1)


def kernel(x, edge_index, W1, a1s, a1d, g1, b1, W2, a2s, a2d, g2, b2,
           W3, a3s, a3d, g3, b3, W4, a4s, a4d, g4, b4,
           tW1, tb1, tW2, tb2, cW1, cb1, cW2, cb2):
    src = edge_index[0].astype(jnp.int32)
    dst = edge_index[1].astype(jnp.int32)
    src2 = src
    dst2 = dst
    src3 = src.reshape(NS, NCH, K)
    dst3 = dst.reshape(NS, NCH, K)
    row = lambda v: v.reshape(1, -1)

    h1, as1, ad1 = pl.pallas_call(
        _tc_first,
        out_shape=[_sds((2 * N, 128)), _sds((N, 1)), _sds((N, 1))],
    )(x, W1, row(a1s), row(a1d))
    acc1, den1 = _edge_layer(h1, as1, ad1, src2, dst2, src3, dst3, 128)

    h2, as2, ad2 = pl.pallas_call(
        _make_tc_mid(256, 0.2),
        out_shape=[_sds((2 * N, 64)), _sds((N, 1)), _sds((N, 1))],
    )(acc1, den1, row(g1), row(b1), W2, row(a2s), row(a2d))
    acc2, den2 = _edge_layer(h2, as2, ad2, src2, dst2, src3, dst3, 64)

    h3, as3, ad3, time_pred, cluster_logits = pl.pallas_call(
        _tc_mid2_heads,
        out_shape=[_sds((2 * N, 128)), _sds((N, 1)), _sds((N, 1)),
                   _sds((N, 1)), _sds((N, 16))],
    )(acc2, den2, row(g2), row(b2), W3, row(a3s), row(a3d),
      tW1, row(tb1), tW2, row(tb2), cW1, row(cb1), cW2, row(cb2))
    acc3, den3 = _edge_layer(h3, as3, ad3, src2, dst2, src3, dst3, 128)

    h4, as4, ad4 = pl.pallas_call(
        _make_tc_mid(256, 0.2),
        out_shape=[_sds((2 * N, 128)), _sds((N, 1)), _sds((N, 1))],
    )(acc3, den3, row(g3), row(b3), W4, row(a4s), row(a4d))
    acc4, den4 = _edge_layer(h4, as4, ad4, src2, dst2, src3, dst3, 128)

    recon = pl.pallas_call(
        _tc_final, out_shape=_sds((N, 256)),
    )(acc4, den4, row(g4), row(b4))

    return recon, time_pred, cluster_logits


# speedup vs baseline: 1.2052x; 1.2052x over previous
import functools

import jax
import jax.numpy as jnp
from jax import lax
from jax.experimental import pallas as pl
from jax.experimental.pallas import tpu as pltpu
from jax.experimental.pallas import tpu_sc as plsc

N = 10000
E = 160000
NS = 16
EPT = E // NS
K = 50
NCH = EPT // K
EG = EPT // 16
RPT = N // NS
RQ = 624

_SC_PARAMS = pltpu.CompilerParams(use_tc_tiling_on_sc=False,
                                  needs_layout_passes=False)
_MESH_KW = dict(core_axis_name="c", subcore_axis_name="s",
                num_cores=2, num_subcores=NS)


@functools.cache
def _alpha_pass():
    ewa = 5008
    ewl = E - 31 * ewa

    @functools.partial(
        pl.kernel,
        mesh=plsc.VectorSubcoreMesh(**_MESH_KW),
        out_type=jax.ShapeDtypeStruct((E,), jnp.float32),
        compiler_params=_SC_PARAMS,
        scratch_types=[
            pltpu.VMEM((N,), jnp.float32),
            pltpu.VMEM((N,), jnp.float32),
            pltpu.VMEM((ewa,), jnp.int32),
            pltpu.VMEM((ewa,), jnp.int32),
            pltpu.VMEM((ewa,), jnp.float32),
        ],
    )
    def alpha(asn_hbm, adn_hbm, src_hbm, dst_hbm, w_hbm,
              asn_v, adn_v, src_f, dst_f, w_f):
        cid = lax.axis_index("c")
        sid = lax.axis_index("s")
        wid = sid * 2 + cid
        base = wid * ewa

        pltpu.sync_copy(asn_hbm, asn_v)
        pltpu.sync_copy(adn_hbm, adn_v)

        @pl.when(wid < 31)
        def _():
            pltpu.sync_copy(src_hbm.at[pl.ds(base, ewa)], src_f)
            pltpu.sync_copy(dst_hbm.at[pl.ds(base, ewa)], dst_f)

        @pl.when(wid == 31)
        def _():
            pltpu.sync_copy(src_hbm.at[pl.ds(base, ewl)],
                            src_f.at[pl.ds(0, ewl)])
            pltpu.sync_copy(dst_hbm.at[pl.ds(base, ewl)],
                            dst_f.at[pl.ds(0, ewl)])

        def wgrp(g, _):
            sv = src_f[pl.ds(g * 16, 16)]
            dv = dst_f[pl.ds(g * 16, 16)]
            al = plsc.load_gather(asn_v, [sv]) + plsc.load_gather(
                adn_v, [dv])
            w_f[pl.ds(g * 16, 16)] = jnp.exp(1.0 / (1.0 + jnp.exp(-al)))
            return 0

        ng = jnp.where(wid == 31, ewl // 16, ewa // 16)
        lax.fori_loop(0, ng, wgrp, 0)

        @pl.when(wid < 31)
        def _():
            pltpu.sync_copy(w_f, w_hbm.at[pl.ds(base, ewa)])

        @pl.when(wid == 31)
        def _():
            pltpu.sync_copy(w_f.at[pl.ds(0, ewl)],
                            w_hbm.at[pl.ds(base, ewl)])

    return alpha


@functools.cache
def _make_agg_pass(dh):
    @functools.partial(
        pl.kernel,
        mesh=plsc.VectorSubcoreMesh(**_MESH_KW),
        out_type=[jax.ShapeDtypeStruct((2 * N, dh), jnp.float32),
                  jax.ShapeDtypeStruct((N,), jnp.float32)],
        compiler_params=_SC_PARAMS,
        scratch_types=[
            pltpu.VMEM((NCH, K), jnp.int32),
            pltpu.VMEM((NCH, K), jnp.int32),
            pltpu.VMEM((K, dh), jnp.float32),
            pltpu.VMEM((K, dh), jnp.float32),
            pltpu.VMEM((K, dh), jnp.float32),
            pltpu.VMEM((K, dh), jnp.float32),
            pltpu.VMEM((K,), jnp.float32),
            pltpu.VMEM((K,), jnp.float32),
            pltpu.VMEM((K,), jnp.float32),
            pltpu.VMEM((K,), jnp.float32),
            pltpu.VMEM((640,), jnp.float32),
            pltpu.VMEM_SHARED((N, dh), jnp.float32),
            pltpu.VMEM_SHARED((N,), jnp.float32),
            pltpu.SemaphoreType.DMA,
            pltpu.SemaphoreType.DMA,
            pltpu.SemaphoreType.DMA,
            pltpu.SemaphoreType.DMA,
            pltpu.SemaphoreType.DMA,
            pltpu.SemaphoreType.DMA,
            pltpu.SemaphoreType.DMA,
            pltpu.SemaphoreType.DMA,
        ],
    )
    def agg(h_hbm, w_hbm, src_hbm, dst_hbm, out_hbm, den_hbm,
            src_v, dst_v, rows_a, rows_b, rows_c, rows_d,
            w_a, w_b, w_c, w_d, zbuf,
            acc, den_acc, g_a, g_b, g_c, g_d, s_a, s_b, s_c, s_d):
        cid = lax.axis_index("c")
        sid = lax.axis_index("s")

        pltpu.sync_copy(src_hbm.at[cid * NS + sid], src_v)
        pltpu.sync_copy(dst_hbm.at[sid], dst_v)
        coff = cid * N

        zeros16 = jnp.zeros((16,), jnp.float32)

        def zrow(r, _):
            for t in range(dh // 16):
                rows_a[r, pl.ds(t * 16, 16)] = zeros16
            return 0

        lax.fori_loop(0, K, zrow, 0)

        def zb(r, _):
            zbuf[pl.ds(r * 16, 16)] = zeros16
            return 0

        lax.fori_loop(0, 640 // 16, zb, 0)

        nz = RPT // K
        for b in range(nz):
            pltpu.sync_copy(rows_a, acc.at[pl.ds(sid * RPT + b * K, K), :])
        rem = RPT - nz * K
        pltpu.sync_copy(rows_a.at[pl.ds(0, rem), :],
                        acc.at[pl.ds(sid * RPT + nz * K, rem), :])
        pltpu.sync_copy(zbuf.at[pl.ds(0, RQ)],
                        den_acc.at[pl.ds(sid * RQ, RQ)])

        @pl.when(sid == NS - 1)
        def _():
            pltpu.sync_copy(zbuf.at[pl.ds(0, N - NS * RQ)],
                            den_acc.at[pl.ds(NS * RQ, N - NS * RQ)])

        plsc.subcore_barrier()

        wrow = sid * NCH

        def start_gather(j, rows_x, w_x, g_x):
            pltpu.async_copy(h_hbm.at[src_v.at[j]], rows_x, g_x)
            pltpu.async_copy(w_hbm.at[wrow + j], w_x, g_x)

        def wait_gather(j, rows_x, w_x, g_x):
            pltpu.make_async_copy(h_hbm.at[src_v.at[j]], rows_x, g_x).wait()
            pltpu.make_async_copy(w_hbm.at[wrow + j], w_x, g_x).wait()

        def scale(rows_x, w_x):
            def srow(e, _):
                wb = plsc.load_gather(
                    w_x, [jnp.full((16,), e, jnp.int32)])
                for t in range(dh // 16):
                    rows_x[e, pl.ds(t * 16, 16)] = (
                        rows_x[e, pl.ds(t * 16, 16)] * wb)
                return 0

            lax.fori_loop(0, K, srow, 0, unroll=4)

        def start_scatter(j, rows_x, w_x, s_x):
            pltpu.async_copy(rows_x, acc.at[dst_v.at[j]], s_x, add=True)
            pltpu.async_copy(w_x, den_acc.at[dst_v.at[j]], s_x, add=True)

        def wait_scatter(j, rows_x, w_x, s_x):
            pltpu.make_async_copy(
                rows_x, acc.at[dst_v.at[j]], s_x).wait()
            pltpu.make_async_copy(
                w_x, den_acc.at[dst_v.at[j]], s_x).wait()

        bufs = [(rows_a, w_a, g_a, s_a),
                (rows_b, w_b, g_b, s_b),
                (rows_c, w_c, g_c, s_c),
                (rows_d, w_d, g_d, s_d)]
        nbuf = 4
        assert NCH % nbuf == 0

        def slot(j, t):
            rows_x, w_x, g_x, s_x = bufs[t]
            wait_gather(j, rows_x, w_x, g_x)
            scale(rows_x, w_x)
            start_scatter(j, rows_x, w_x, s_x)
            ry, wy, gy, sy = bufs[(t + nbuf - 1) % nbuf]

            @pl.when(j + nbuf - 1 < NCH)
            def _():
                @pl.when(j >= 1)
                def _():
                    wait_scatter(j - 1, ry, wy, sy)

                start_gather(j + nbuf - 1, ry, wy, gy)

        for j in range(nbuf - 1):
            rx, wx, gx, sx = bufs[j]
            start_gather(j, rx, wx, gx)

        def body(m, _):
            j0 = nbuf * m
            for t in range(nbuf):
                slot(j0 + t, t)
            return 0

        lax.fori_loop(0, NCH // nbuf, body, 0)
        for j in range(NCH - nbuf, NCH):
            rx, wx, gx, sx = bufs[j % nbuf]
            wait_scatter(j, rx, wx, sx)
        plsc.subcore_barrier()
        pltpu.sync_copy(acc.at[pl.ds(sid * RPT, RPT), :],
                        out_hbm.at[pl.ds(coff + sid * RPT, RPT), :])

        @pl.when(cid == 0)
        def _():
            pltpu.sync_copy(den_acc.at[pl.ds(sid * RQ, RQ)],
                            den_hbm.at[pl.ds(sid * RQ, RQ)])

            @pl.when(sid == NS - 1)
            def _():
                pltpu.sync_copy(den_acc.at[pl.ds(NS * RQ, N - NS * RQ)],
                                den_hbm.at[pl.ds(NS * RQ, N - NS * RQ)])

    return agg


def _hext(h, d):
    dh = d // 2
    return jnp.concatenate([h[:, :dh], h[:, dh:]], axis=0)


def _unpack_norm(acc, den, d):
    dh = d // 2
    inv = 1.0 / (den + 1e-16)
    return jnp.concatenate([acc[:N, :dh] * inv, acc[N:, :dh] * inv], axis=1)


def _bn_body(y, g, b):
    mu = jnp.mean(y, axis=0, keepdims=True)
    yc = y - mu
    var = jnp.mean(yc * yc, axis=0, keepdims=True)
    return yc * lax.rsqrt(var + 1e-5) * g + b


def _leaky(x, slope):
    return jnp.where(x >= 0, x, slope * x)


def _tc_first(x_ref, w_ref, as_ref, ad_ref, hext_ref, asn_ref, adn_ref):
    h = jnp.dot(x_ref[...], w_ref[...], preferred_element_type=jnp.float32)
    asn_ref[...] = jnp.sum(h * as_ref[...], axis=1, keepdims=True)
    adn_ref[...] = jnp.sum(h * ad_ref[...], axis=1, keepdims=True)
    hext_ref[...] = _hext(h, w_ref.shape[1])


def _make_tc_mid(d, slope):
    def body(acc_ref, den_ref, g_ref, b_ref, w_ref, as_ref, ad_ref,
             hext_ref, asn_ref, adn_ref):
        y = _unpack_norm(acc_ref[...], den_ref[...], d)
        ybn = _bn_body(y, g_ref[...], b_ref[...])
        if slope is not None:
            ybn = _leaky(ybn, slope)
        h = jnp.dot(ybn, w_ref[...], preferred_element_type=jnp.float32)
        asn_ref[...] = jnp.sum(h * as_ref[...], axis=1, keepdims=True)
        adn_ref[...] = jnp.sum(h * ad_ref[...], axis=1, keepdims=True)
        hext_ref[...] = _hext(h, w_ref.shape[1])
    return body


def _tc_mid2_heads(acc_ref, den_ref, g_ref, b_ref, w_ref, as_ref, ad_ref,
                   tw1_ref, tb1_ref, tw2_ref, tb2_ref,
                   cw1_ref, cb1_ref, cw2_ref, cb2_ref,
                   hext_ref, asn_ref, adn_ref, tp_ref, cl_ref):
    z = _bn_body(_unpack_norm(acc_ref[...], den_ref[...], 128),
                 g_ref[...], b_ref[...])
    h = jnp.dot(z, w_ref[...], preferred_element_type=jnp.float32)
    asn_ref[...] = jnp.sum(h * as_ref[...], axis=1, keepdims=True)
    adn_ref[...] = jnp.sum(h * ad_ref[...], axis=1, keepdims=True)
    hext_ref[...] = _hext(h, w_ref.shape[1])
    t = _leaky(jnp.dot(z, tw1_ref[...]) + tb1_ref[...], 0.01)
    tp_ref[...] = jax.nn.sigmoid(jnp.dot(t, tw2_ref[...]) + tb2_ref[...])
    c = _leaky(jnp.dot(z, cw1_ref[...]) + cb1_ref[...], 0.01)
    cl_ref[...] = jnp.dot(c, cw2_ref[...]) + cb2_ref[...]


def _tc_final(acc_ref, den_ref, g_ref, b_ref, out_ref):
    out_ref[...] = _bn_body(_unpack_norm(acc_ref[...], den_ref[...], 256),
                            g_ref[...], b_ref[...])


def _sds(shape):
    return jax.ShapeDtypeStruct(shape, jnp.float32)


def _edge_layer(hext, asn, adn, src2, dst2, src3, dst3, dh):
    w = _alpha_pass()(asn.reshape(N), adn.reshape(N), src2, dst2)
    acc, den = _make_agg_pass(dh)(hext, w.reshape(NS * NCH, K), src3, dst3)
    return acc, den.reshape(N, 1)


def kernel(x, edge_index, W1, a1s, a1d, g1, b1, W2, a2s, a2d, g2, b2,
           W3, a3s, a3d, g3, b3, W4, a4s, a4d, g4, b4,
           tW1, tb1, tW2, tb2, cW1, cb1, cW2, cb2):
    src = edge_index[0].astype(jnp.int32)
    dst = edge_index[1].astype(jnp.int32)
    src2 = src
    dst2 = dst
    src3 = src.reshape(NS, NCH, K)
    src3 = jnp.concatenate([src3, src3 + N], axis=0)
    dst3 = dst.reshape(NS, NCH, K)
    row = lambda v: v.reshape(1, -1)

    h1, as1, ad1 = pl.pallas_call(
        _tc_first,
        out_shape=[_sds((2 * N, 128)), _sds((N, 1)), _sds((N, 1))],
    )(x, W1, row(a1s), row(a1d))
    acc1, den1 = _edge_layer(h1, as1, ad1, src2, dst2, src3, dst3, 128)

    h2, as2, ad2 = pl.pallas_call(
        _make_tc_mid(256, 0.2),
        out_shape=[_sds((2 * N, 64)), _sds((N, 1)), _sds((N, 1))],
    )(acc1, den1, row(g1), row(b1), W2, row(a2s), row(a2d))
    acc2, den2 = _edge_layer(h2, as2, ad2, src2, dst2, src3, dst3, 64)

    h3, as3, ad3, time_pred, cluster_logits = pl.pallas_call(
        _tc_mid2_heads,
        out_shape=[_sds((2 * N, 128)), _sds((N, 1)), _sds((N, 1)),
                   _sds((N, 1)), _sds((N, 16))],
    )(acc2, den2, row(g2), row(b2), W3, row(a3s), row(a3d),
      tW1, row(tb1), tW2, row(tb2), cW1, row(cb1), cW2, row(cb2))
    acc3, den3 = _edge_layer(h3, as3, ad3, src2, dst2, src3, dst3, 128)

    h4, as4, ad4 = pl.pallas_call(
        _make_tc_mid(256, 0.2),
        out_shape=[_sds((2 * N, 128)), _sds((N, 1)), _sds((N, 1))],
    )(acc3, den3, row(g3), row(b3), W4, row(a4s), row(a4d))
    acc4, den4 = _edge_layer(h4, as4, ad4, src2, dst2, src3, dst3, 128)

    recon = pl.pallas_call(
        _tc_final, out_shape=_sds((N, 256)),
    )(acc4, den4, row(g4), row(b4))

    return recon, time_pred, cluster_logits
